# baseline (speedup 1.0000x reference)
import jax
import jax.numpy as jnp
from jax.experimental import pallas as pl
from jax.experimental.pallas import tpu as pltpu

BATCH = 64
NUM_PATCHES = 576
PROJ = 768


def _add_body(enc_ref, pos_ref, out_ref):
    out_ref[...] = enc_ref[...] + pos_ref[...]


def kernel(encoded_patches, position_table):
    grid = (BATCH,)
    return pl.pallas_call(
        _add_body,
        grid=grid,
        in_specs=[
            pl.BlockSpec((1, NUM_PATCHES, PROJ), lambda b: (b, 0, 0)),
            pl.BlockSpec((NUM_PATCHES, PROJ), lambda b: (0, 0)),
        ],
        out_specs=pl.BlockSpec((1, NUM_PATCHES, PROJ), lambda b: (b, 0, 0)),
        out_shape=jax.ShapeDtypeStruct((BATCH, NUM_PATCHES, PROJ), jnp.float32),
    )(encoded_patches, position_table)


# speedup vs baseline: 1.1889x; 1.1889x over previous
import jax
import jax.numpy as jnp
from jax.experimental import pallas as pl
from jax.experimental.pallas import tpu as pltpu

BATCH = 64
NUM_PATCHES = 576
PROJ = 768


def _add_body(enc_ref, pos_ref, out_ref):
    out_ref[...] = enc_ref[...] + pos_ref[...]


def kernel(encoded_patches, position_table):
    bb = 4
    grid = (BATCH // bb,)
    return pl.pallas_call(
        _add_body,
        grid=grid,
        in_specs=[
            pl.BlockSpec((bb, NUM_PATCHES, PROJ), lambda b: (b, 0, 0)),
            pl.BlockSpec((NUM_PATCHES, PROJ), lambda b: (0, 0)),
        ],
        out_specs=pl.BlockSpec((bb, NUM_PATCHES, PROJ), lambda b: (b, 0, 0)),
        out_shape=jax.ShapeDtypeStruct((BATCH, NUM_PATCHES, PROJ), jnp.float32),
    )(encoded_patches, position_table)
